# Initial kernel scaffold; baseline (speedup 1.0000x reference)
import jax
import jax.numpy as jnp
from jax.experimental import pallas as pl


def kernel(x, chan_ind, spat_ind, embed_channel, embed_spatial):
    raise NotImplementedError("write your pallas kernel here")


# speedup vs baseline: 1.8754x; 1.8754x over previous
import functools

import jax
import jax.numpy as jnp
from jax import lax
from jax.experimental import pallas as pl
from jax.experimental.pallas import tpu as pltpu
from jax.experimental.pallas import tpu_sc as plsc

N = 16384
HID = 1024
D = 256
NC, NS, L = 2, 16, 16
NW = NC * NS
ROWS_W = N // NW
CHUNK = 64
NCHUNK = ROWS_W // CHUNK


def _sc_add_embed(x, c0, c1, s0, s1, tab_c, tab_s):
    mesh = plsc.VectorSubcoreMesh(core_axis_name="c", subcore_axis_name="s")

    @functools.partial(
        pl.kernel,
        mesh=mesh,
        out_type=jax.ShapeDtypeStruct((N, HID), jnp.float32),
        scratch_types=[
            pltpu.VMEM((ROWS_W,), jnp.int32),
            pltpu.VMEM((ROWS_W,), jnp.int32),
            pltpu.VMEM((ROWS_W,), jnp.int32),
            pltpu.VMEM((ROWS_W,), jnp.int32),
            pltpu.VMEM((CHUNK, D), jnp.float32),
            pltpu.VMEM((CHUNK, D), jnp.float32),
            pltpu.SemaphoreType.DMA,
        ],
    )
    def k(x_hbm, c0_hbm, c1_hbm, s0_hbm, s1_hbm, tc_hbm, ts_hbm, out_hbm,
          i0_v, i1_v, i2_v, i3_v, xbuf, gbuf, sem):
        wid = lax.axis_index("s") * NC + lax.axis_index("c")
        base = wid * ROWS_W
        pltpu.sync_copy(c0_hbm.at[pl.ds(base, ROWS_W)], i0_v)
        pltpu.sync_copy(c1_hbm.at[pl.ds(base, ROWS_W)], i1_v)
        pltpu.sync_copy(s0_hbm.at[pl.ds(base, ROWS_W)], i2_v)
        pltpu.sync_copy(s1_hbm.at[pl.ds(base, ROWS_W)], i3_v)

        for p, tab, idx in ((0, tc_hbm, i0_v), (1, tc_hbm, i1_v),
                            (2, ts_hbm, i2_v), (3, ts_hbm, i3_v)):
            def chunk_body(ci, _, p=p, tab=tab, idx=idx):
                r0 = base + ci * CHUNK
                pltpu.sync_copy(
                    x_hbm.at[pl.ds(r0, CHUNK), pl.ds(p * D, D)], xbuf)
                pltpu.async_copy(
                    tab.at[idx.at[pl.ds(ci * CHUNK, CHUNK)]], gbuf, sem
                ).wait()

                def row_body(i, _):
                    for j in range(D // L):
                        g = gbuf[i, pl.ds(j * L, L)]
                        plsc.addupdate(xbuf.at[i, pl.ds(j * L, L)], g)
                    return 0

                lax.fori_loop(0, CHUNK, row_body, 0)
                pltpu.sync_copy(
                    xbuf, out_hbm.at[pl.ds(r0, CHUNK), pl.ds(p * D, D)])
                return 0

            lax.fori_loop(0, NCHUNK, chunk_body, 0)

    return k(x, c0, c1, s0, s1, tab_c, tab_s)


def kernel(x, chan_ind, spat_ind, embed_channel, embed_spatial):
    c0 = chan_ind[:, 0].astype(jnp.int32)
    c1 = chan_ind[:, 1].astype(jnp.int32)
    s0 = spat_ind[:, 0].astype(jnp.int32)
    s1 = spat_ind[:, 1].astype(jnp.int32)
    return _sc_add_embed(x, c0, c1, s0, s1,
                         embed_channel.astype(jnp.float32),
                         embed_spatial.astype(jnp.float32))
